# Initial kernel scaffold; baseline (speedup 1.0000x reference)
import jax
import jax.numpy as jnp
from jax.experimental import pallas as pl


def kernel(atom_fea, edge_fea, sub_atom_idx, sub_edge_idx, sub_edge_ang, sub_index, distance, huge_structure, output_final_layer_neuron, W_f, b_f, W_s, b_s, W_e1, b_e1, W_e2, b_e2):
    raise NotImplementedError("write your pallas kernel here")


# speedup vs baseline: 5.1542x; 5.1542x over previous
import functools

import jax
import jax.numpy as jnp
from jax import lax
from jax.experimental import pallas as pl
from jax.experimental.pallas import tpu as pltpu
from jax.experimental.pallas import tpu_sc as plsc

N_NODES = 10000
N_EDGES = 160000
M = 2 * N_EDGES
A = 128
E_FEAT = 112
ANG = 16
ZDIM = 384

NC = 2
NS = 16
NW = NC * NS
PER_W = M // NW
CHUNK = 200
NSTEP = PER_W // CHUNK


def _sc_assemble(atom_fea, edge_aug, i0, i1, ij):
    mesh = plsc.VectorSubcoreMesh(core_axis_name="c", subcore_axis_name="s")

    @functools.partial(
        pl.kernel,
        out_type=jax.ShapeDtypeStruct((M, ZDIM), jnp.float32),
        mesh=mesh,
        scratch_types=[
            pltpu.VMEM((CHUNK,), jnp.int32),
            pltpu.VMEM((CHUNK,), jnp.int32),
            pltpu.VMEM((CHUNK,), jnp.int32),
            pltpu.VMEM((CHUNK, A), jnp.float32),
            pltpu.VMEM((CHUNK, A), jnp.float32),
            pltpu.VMEM((CHUNK, A), jnp.float32),
            pltpu.SemaphoreType.DMA,
            pltpu.SemaphoreType.DMA,
            pltpu.SemaphoreType.DMA,
        ],
    )
    def k(atom_hbm, edge_hbm, i0_hbm, i1_hbm, ij_hbm,
          z_hbm,
          idx0_v, idx1_v, idxj_v, b0, b1, b2,
          sem0, sem1, sem2):
        wid = lax.axis_index("s") * NC + lax.axis_index("c")
        base0 = wid * PER_W

        def step(i, carry):
            base = base0 + i * CHUNK
            rows = pl.ds(base, CHUNK)
            pltpu.sync_copy(i0_hbm.at[rows], idx0_v)
            pltpu.sync_copy(i1_hbm.at[rows], idx1_v)
            pltpu.sync_copy(ij_hbm.at[rows], idxj_v)
            cp0 = pltpu.async_copy(atom_hbm.at[idx0_v], b0, sem0)
            cp1 = pltpu.async_copy(atom_hbm.at[idx1_v], b1, sem1)
            cp2 = pltpu.async_copy(edge_hbm.at[idxj_v], b2, sem2)
            cp0.wait()
            cp1.wait()
            cp2.wait()
            pltpu.sync_copy(b0, z_hbm.at[rows, pl.ds(0, A)])
            pltpu.sync_copy(b1, z_hbm.at[rows, pl.ds(A, A)])
            pltpu.sync_copy(b2, z_hbm.at[rows, pl.ds(2 * A, A)])
            return carry

        lax.fori_loop(0, NSTEP, step, 0)

    return k(atom_fea, edge_aug, i0, i1, ij)


BE = 640
DCOL = 2 * A + E_FEAT


def _tc_body(zp_ref, ap_ref, ef_ref, wfs_ref, wang_ref, bfs_ref,
             we1_ref, be1_ref, we2_ref, be2_ref, out_ref):
    zp = zp_ref[...]
    x = jnp.concatenate([zp[:, :ZDIM], zp[:, ZDIM:]], axis=0)
    ap = ap_ref[...]
    xang = jnp.concatenate([ap[:, :ANG], ap[:, ANG:]], axis=0)
    zz = jnp.dot(x, wfs_ref[...], preferred_element_type=jnp.float32)
    zz = zz + jnp.dot(xang, wang_ref[...], preferred_element_type=jnp.float32)
    zz = zz + bfs_ref[...]
    d = x[:, DCOL:DCOL + 1]
    expd = jnp.exp(d * d * (-1.0 / 18.0))
    g = jax.nn.sigmoid(zz[:, :A]) * jax.nn.softplus(zz[:, A:]) * expd
    cat = jnp.concatenate([g[:BE], g[BE:], ef_ref[...]], axis=-1)
    h = jnp.dot(cat, we1_ref[...], preferred_element_type=jnp.float32)
    h = jax.nn.silu(h + be1_ref[...])
    o = jnp.dot(h, we2_ref[...], preferred_element_type=jnp.float32)
    out_ref[...] = o + be2_ref[...]


def _tc_compute(zp, ap, edge_fea, w_fs, w_ang, b_fs, w_e1, b_e1, w_e2, b_e2):
    nblk = N_EDGES // BE
    full = lambda shape: pl.BlockSpec(shape, lambda i: (0, 0))
    return pl.pallas_call(
        _tc_body,
        grid=(nblk,),
        in_specs=[
            pl.BlockSpec((BE, 2 * ZDIM), lambda i: (i, 0)),
            pl.BlockSpec((BE, 2 * ANG), lambda i: (i, 0)),
            pl.BlockSpec((BE, E_FEAT), lambda i: (i, 0)),
            full(w_fs.shape),
            full(w_ang.shape),
            full(b_fs.shape),
            full(w_e1.shape),
            full(b_e1.shape),
            full(w_e2.shape),
            full(b_e2.shape),
        ],
        out_specs=pl.BlockSpec((BE, 64), lambda i: (i, 0)),
        out_shape=jax.ShapeDtypeStruct((N_EDGES, 64), jnp.float32),
        compiler_params=pltpu.CompilerParams(
            dimension_semantics=("arbitrary",),
        ),
    )(zp, ap, edge_fea, w_fs, w_ang, b_fs, w_e1, b_e1, w_e2, b_e2)


def kernel(atom_fea, edge_fea, sub_atom_idx, sub_edge_idx, sub_edge_ang,
           sub_index, distance, huge_structure, output_final_layer_neuron,
           W_f, b_f, W_s, b_s, W_e1, b_e1, W_e2, b_e2):
    i0 = sub_atom_idx[:, 0].astype(jnp.int32)
    i1 = sub_atom_idx[:, 1].astype(jnp.int32)
    ij = sub_edge_idx.astype(jnp.int32)
    edge_aug = jnp.concatenate(
        [edge_fea, distance[:, None],
         jnp.zeros((N_EDGES, A - E_FEAT - 1), jnp.float32)], axis=1)
    zfull = _sc_assemble(atom_fea, edge_aug, i0, i1, ij)
    zp = zfull.reshape(N_EDGES, 2 * ZDIM)
    ap = sub_edge_ang.reshape(N_EDGES, 2 * ANG)
    w_fs = jnp.concatenate([W_f, W_s], axis=1)
    w_fs_pad = w_fs.at[DCOL:, :].set(0.0)
    w_ang = w_fs[ZDIM - ANG:, :]
    b_fs = jnp.concatenate([b_f, b_s])[None, :]
    return _tc_compute(zp, ap, edge_fea, w_fs_pad, w_ang, b_fs,
                       W_e1, b_e1[None, :], W_e2, b_e2[None, :])


# speedup vs baseline: 5.9719x; 1.1587x over previous
import functools

import jax
import jax.numpy as jnp
from jax import lax
from jax.experimental import pallas as pl
from jax.experimental.pallas import tpu as pltpu
from jax.experimental.pallas import tpu_sc as plsc

N_NODES = 10000
N_EDGES = 160000
M = 2 * N_EDGES
A = 128
E_FEAT = 112
ANG = 16
ZDIM = 384

NC = 2
NS = 16
NW = NC * NS
CE = 128
NCHK = N_EDGES // CE


def _sc_assemble(atom_fea, edge_aug, i0e, i1e, ije, i0o, i1o, ijo):
    mesh = plsc.VectorSubcoreMesh(core_axis_name="c", subcore_axis_name="s")

    @functools.partial(
        pl.kernel,
        out_type=jax.ShapeDtypeStruct((N_EDGES, 6 * A), jnp.float32),
        mesh=mesh,
        scratch_types=[
            pltpu.VMEM((CE,), jnp.int32),
            pltpu.VMEM((CE,), jnp.int32),
            pltpu.VMEM((CE,), jnp.int32),
            pltpu.VMEM((CE,), jnp.int32),
            pltpu.VMEM((CE,), jnp.int32),
            pltpu.VMEM((CE,), jnp.int32),
            pltpu.VMEM((CE, A), jnp.float32),
            pltpu.VMEM((CE, A), jnp.float32),
            pltpu.VMEM((CE, A), jnp.float32),
            pltpu.VMEM((CE, A), jnp.float32),
            pltpu.VMEM((CE, A), jnp.float32),
            pltpu.VMEM((CE, A), jnp.float32),
            pltpu.SemaphoreType.DMA,
            pltpu.SemaphoreType.DMA,
            pltpu.SemaphoreType.DMA,
            pltpu.SemaphoreType.DMA,
            pltpu.SemaphoreType.DMA,
            pltpu.SemaphoreType.DMA,
        ],
    )
    def k(atom_hbm, edge_hbm, i0e_h, i1e_h, ije_h, i0o_h, i1o_h, ijo_h,
          z_hbm,
          x0, x1, x2, x3, x4, x5, b0, b1, b2, b3, b4, b5,
          s0, s1, s2, s3, s4, s5):
        wid = lax.axis_index("s") * NC + lax.axis_index("c")
        nsteps = (NCHK - 1 - wid) // NW + 1

        def step(t, carry):
            c = wid + t * NW
            rows = pl.ds(c * CE, CE)
            pltpu.sync_copy(i0e_h.at[rows], x0)
            pltpu.sync_copy(i1e_h.at[rows], x1)
            pltpu.sync_copy(ije_h.at[rows], x2)
            pltpu.sync_copy(i0o_h.at[rows], x3)
            pltpu.sync_copy(i1o_h.at[rows], x4)
            pltpu.sync_copy(ijo_h.at[rows], x5)
            cps = [
                pltpu.async_copy(atom_hbm.at[x0], b0, s0),
                pltpu.async_copy(atom_hbm.at[x1], b1, s1),
                pltpu.async_copy(edge_hbm.at[x2], b2, s2),
                pltpu.async_copy(atom_hbm.at[x3], b3, s3),
                pltpu.async_copy(atom_hbm.at[x4], b4, s4),
                pltpu.async_copy(edge_hbm.at[x5], b5, s5),
            ]
            for cp in cps:
                cp.wait()
            pltpu.sync_copy(b0, z_hbm.at[rows, pl.ds(0, A)])
            pltpu.sync_copy(b1, z_hbm.at[rows, pl.ds(A, A)])
            pltpu.sync_copy(b2, z_hbm.at[rows, pl.ds(2 * A, A)])
            pltpu.sync_copy(b3, z_hbm.at[rows, pl.ds(3 * A, A)])
            pltpu.sync_copy(b4, z_hbm.at[rows, pl.ds(4 * A, A)])
            pltpu.sync_copy(b5, z_hbm.at[rows, pl.ds(5 * A, A)])
            return carry

        lax.fori_loop(0, nsteps, step, 0)

    return k(atom_fea, edge_aug, i0e, i1e, ije, i0o, i1o, ijo)


BE = 640
DCOL = 2 * A + E_FEAT


def _tc_body(zp_ref, ang_ref, ef_ref, wfs_ref, wang_ref, bfs_ref,
             we1_ref, be1_ref, we2_ref, be2_ref, out_ref):
    zp = zp_ref[...]
    x = jnp.concatenate([zp[:, :ZDIM], zp[:, ZDIM:]], axis=0)
    ap = ang_ref[...]
    xang = jnp.concatenate([ap[:, :ANG], ap[:, ANG:]], axis=0)
    zz = jnp.dot(x, wfs_ref[...], preferred_element_type=jnp.float32)
    zz = zz + jnp.dot(xang, wang_ref[...], preferred_element_type=jnp.float32)
    zz = zz + bfs_ref[...]
    d = x[:, DCOL:DCOL + 1]
    expd = jnp.exp(d * d * (-1.0 / 18.0))
    g = jax.nn.sigmoid(zz[:, :A]) * jax.nn.softplus(zz[:, A:]) * expd
    cat = jnp.concatenate([g[:BE], g[BE:], ef_ref[...]], axis=-1)
    h = jnp.dot(cat, we1_ref[...], preferred_element_type=jnp.float32)
    h = jax.nn.silu(h + be1_ref[...])
    o = jnp.dot(h, we2_ref[...], preferred_element_type=jnp.float32)
    out_ref[...] = o + be2_ref[...]


def _tc_compute(zp, ang, edge_fea, w_fs, w_ang, b_fs, w_e1, b_e1, w_e2, b_e2):
    nblk = N_EDGES // BE
    full = lambda shape: pl.BlockSpec(shape, lambda i: (0, 0))
    return pl.pallas_call(
        _tc_body,
        grid=(nblk,),
        in_specs=[
            pl.BlockSpec((BE, 6 * A), lambda i: (i, 0)),
            pl.BlockSpec((BE, 2 * ANG), lambda i: (i, 0)),
            pl.BlockSpec((BE, E_FEAT), lambda i: (i, 0)),
            full(w_fs.shape),
            full(w_ang.shape),
            full(b_fs.shape),
            full(w_e1.shape),
            full(b_e1.shape),
            full(w_e2.shape),
            full(b_e2.shape),
        ],
        out_specs=pl.BlockSpec((BE, 64), lambda i: (i, 0)),
        out_shape=jax.ShapeDtypeStruct((N_EDGES, 64), jnp.float32),
        compiler_params=pltpu.CompilerParams(
            dimension_semantics=("arbitrary",),
        ),
    )(zp, ang, edge_fea, w_fs, w_ang, b_fs, w_e1, b_e1, w_e2, b_e2)


def kernel(atom_fea, edge_fea, sub_atom_idx, sub_edge_idx, sub_edge_ang,
           sub_index, distance, huge_structure, output_final_layer_neuron,
           W_f, b_f, W_s, b_s, W_e1, b_e1, W_e2, b_e2):
    sai = sub_atom_idx.astype(jnp.int32)
    ij = sub_edge_idx.astype(jnp.int32)
    i0e = sai[0::2, 0]
    i1e = sai[0::2, 1]
    ije = ij[0::2]
    i0o = sai[1::2, 0]
    i1o = sai[1::2, 1]
    ijo = ij[1::2]
    edge_aug = jnp.concatenate(
        [edge_fea, distance[:, None],
         jnp.zeros((N_EDGES, A - E_FEAT - 1), jnp.float32)], axis=1)
    zp = _sc_assemble(atom_fea, edge_aug, i0e, i1e, ije, i0o, i1o, ijo)
    w_fs = jnp.concatenate([W_f, W_s], axis=1)
    w_fs_pad = w_fs.at[DCOL:, :].set(0.0)
    w_ang = w_fs[ZDIM - ANG:, :]
    b_fs = jnp.concatenate([b_f, b_s])[None, :]
    ap = sub_edge_ang.reshape(N_EDGES, 2 * ANG)
    return _tc_compute(zp, ap, edge_fea, w_fs_pad, w_ang, b_fs,
                       W_e1, b_e1[None, :], W_e2, b_e2[None, :])


# speedup vs baseline: 6.4483x; 1.0798x over previous
import functools

import jax
import jax.numpy as jnp
from jax import lax
from jax.experimental import pallas as pl
from jax.experimental.pallas import tpu as pltpu
from jax.experimental.pallas import tpu_sc as plsc

N_NODES = 10000
N_EDGES = 160000
M = 2 * N_EDGES
A = 128
E_FEAT = 112
ANG = 16
ZDIM = 384

NC = 2
NS = 16
NW = NC * NS
CE = 64
NCHK = N_EDGES // CE
NT = (NCHK + NW - 1) // NW
NPAIR = (NT + 1) // 2


def _sc_assemble(atom_fea, edge_aug, idx_pack):
    mesh = plsc.VectorSubcoreMesh(core_axis_name="c", subcore_axis_name="s")

    @functools.partial(
        pl.kernel,
        out_type=jax.ShapeDtypeStruct((N_EDGES, 6 * A), jnp.float32),
        mesh=mesh,
        scratch_types=[
            pltpu.VMEM((6 * CE,), jnp.int32),
            pltpu.VMEM((6 * CE,), jnp.int32),
            pltpu.VMEM((CE, A), jnp.float32),
            pltpu.VMEM((CE, A), jnp.float32),
            pltpu.VMEM((CE, A), jnp.float32),
            pltpu.VMEM((CE, A), jnp.float32),
            pltpu.VMEM((CE, A), jnp.float32),
            pltpu.VMEM((CE, A), jnp.float32),
            pltpu.VMEM((CE, A), jnp.float32),
            pltpu.VMEM((CE, A), jnp.float32),
            pltpu.VMEM((CE, A), jnp.float32),
            pltpu.VMEM((CE, A), jnp.float32),
            pltpu.VMEM((CE, A), jnp.float32),
            pltpu.VMEM((CE, A), jnp.float32),
            pltpu.SemaphoreType.DMA,
            pltpu.SemaphoreType.DMA,
            pltpu.SemaphoreType.DMA,
            pltpu.SemaphoreType.DMA,
        ],
    )
    def k(atom_hbm, edge_hbm, idx_hbm,
          z_hbm,
          xa0, xa1,
          b00, b01, b02, b03, b04, b05,
          b10, b11, b12, b13, b14, b15,
          sg0, sg1, sw0, sw1):
        wid = lax.axis_index("s") * NC + lax.axis_index("c")
        xall = (xa0, xa1)
        bufs = ((b00, b01, b02, b03, b04, b05),
                (b10, b11, b12, b13, b14, b15))
        sg = (sg0, sg1)
        sw = (sw0, sw1)
        tabs = (atom_hbm, atom_hbm, edge_hbm, atom_hbm, atom_hbm, edge_hbm)

        def chunk_of(t):
            return wid + t * NW

        def cond(t):
            return chunk_of(t) < NCHK

        def gathers_start(t, s):
            c = chunk_of(t)
            pltpu.sync_copy(idx_hbm.at[pl.ds(c * 6 * CE, 6 * CE)], xall[s])
            for kk in range(6):
                pltpu.async_copy(
                    tabs[kk].at[xall[s].at[pl.ds(kk * CE, CE)]],
                    bufs[s][kk], sg[s])

        def gathers_wait(s):
            for kk in range(6):
                pltpu.make_async_copy(
                    tabs[kk].at[xall[s].at[pl.ds(kk * CE, CE)]],
                    bufs[s][kk], sg[s]).wait()

        def writes_start(t, s):
            rows = pl.ds(chunk_of(t) * CE, CE)
            for kk in range(6):
                pltpu.async_copy(bufs[s][kk],
                                 z_hbm.at[rows, pl.ds(kk * A, A)], sw[s])

        def writes_wait(s):
            rows = pl.ds(0, CE)
            for kk in range(6):
                pltpu.make_async_copy(bufs[s][kk],
                                      z_hbm.at[rows, pl.ds(kk * A, A)],
                                      sw[s]).wait()

        gathers_start(0, 0)

        def pair(tt, carry):
            t0 = 2 * tt
            t1 = t0 + 1
            t2 = t0 + 2
            @pl.when(jnp.logical_and(cond(t1), t1 >= 3))
            def _():
                writes_wait(1)

            @pl.when(cond(t1))
            def _():
                gathers_start(t1, 1)

            @pl.when(cond(t0))
            def _():
                gathers_wait(0)
                writes_start(t0, 0)

            @pl.when(cond(t2))
            def _():
                writes_wait(0)
                gathers_start(t2, 0)

            @pl.when(cond(t1))
            def _():
                gathers_wait(1)
                writes_start(t1, 1)

            return carry

        lax.fori_loop(0, NPAIR, pair, 0)
        writes_wait(0)
        writes_wait(1)

    return k(atom_fea, edge_aug, idx_pack)


BE = 640
DCOL = 2 * A + E_FEAT


def _tc_body(zp_ref, ang_ref, ef_ref, wfs_ref, wang_ref, bfs_ref,
             we1_ref, be1_ref, we2_ref, be2_ref, out_ref):
    zp = zp_ref[...]
    x = jnp.concatenate([zp[:, :ZDIM], zp[:, ZDIM:]], axis=0)
    ap = ang_ref[...]
    xang = jnp.concatenate([ap[:, :ANG], ap[:, ANG:]], axis=0)
    zz = jnp.dot(x, wfs_ref[...], preferred_element_type=jnp.float32)
    zz = zz + jnp.dot(xang, wang_ref[...], preferred_element_type=jnp.float32)
    zz = zz + bfs_ref[...]
    d = x[:, DCOL:DCOL + 1]
    expd = jnp.exp(d * d * (-1.0 / 18.0))
    g = jax.nn.sigmoid(zz[:, :A]) * jax.nn.softplus(zz[:, A:]) * expd
    cat = jnp.concatenate([g[:BE], g[BE:], ef_ref[...]], axis=-1)
    h = jnp.dot(cat, we1_ref[...], preferred_element_type=jnp.float32)
    h = jax.nn.silu(h + be1_ref[...])
    o = jnp.dot(h, we2_ref[...], preferred_element_type=jnp.float32)
    out_ref[...] = o + be2_ref[...]


def _tc_compute(zp, ang, edge_fea, w_fs, w_ang, b_fs, w_e1, b_e1, w_e2, b_e2):
    nblk = N_EDGES // BE
    full = lambda shape: pl.BlockSpec(shape, lambda i: (0, 0))
    return pl.pallas_call(
        _tc_body,
        grid=(nblk,),
        in_specs=[
            pl.BlockSpec((BE, 6 * A), lambda i: (i, 0)),
            pl.BlockSpec((BE, 2 * ANG), lambda i: (i, 0)),
            pl.BlockSpec((BE, E_FEAT), lambda i: (i, 0)),
            full(w_fs.shape),
            full(w_ang.shape),
            full(b_fs.shape),
            full(w_e1.shape),
            full(b_e1.shape),
            full(w_e2.shape),
            full(b_e2.shape),
        ],
        out_specs=pl.BlockSpec((BE, 64), lambda i: (i, 0)),
        out_shape=jax.ShapeDtypeStruct((N_EDGES, 64), jnp.float32),
        compiler_params=pltpu.CompilerParams(
            dimension_semantics=("arbitrary",),
        ),
    )(zp, ang, edge_fea, w_fs, w_ang, b_fs, w_e1, b_e1, w_e2, b_e2)


def kernel(atom_fea, edge_fea, sub_atom_idx, sub_edge_idx, sub_edge_ang,
           sub_index, distance, huge_structure, output_final_layer_neuron,
           W_f, b_f, W_s, b_s, W_e1, b_e1, W_e2, b_e2):
    sai = sub_atom_idx.astype(jnp.int32)
    ij = sub_edge_idx.astype(jnp.int32)
    i0e = sai[0::2, 0]
    i1e = sai[0::2, 1]
    ije = ij[0::2]
    i0o = sai[1::2, 0]
    i1o = sai[1::2, 1]
    ijo = ij[1::2]
    idx_pack = (jnp.stack([i0e, i1e, ije, i0o, i1o, ijo], axis=0)
                .reshape(6, NCHK, CE).transpose(1, 0, 2).reshape(-1))
    edge_aug = jnp.concatenate(
        [edge_fea, distance[:, None],
         jnp.zeros((N_EDGES, A - E_FEAT - 1), jnp.float32)], axis=1)
    zp = _sc_assemble(atom_fea, edge_aug, idx_pack)
    w_fs = jnp.concatenate([W_f, W_s], axis=1)
    w_fs_pad = w_fs.at[DCOL:, :].set(0.0)
    w_ang = w_fs[ZDIM - ANG:, :]
    b_fs = jnp.concatenate([b_f, b_s])[None, :]
    ap = sub_edge_ang.reshape(N_EDGES, 2 * ANG)
    return _tc_compute(zp, ap, edge_fea, w_fs_pad, w_ang, b_fs,
                       W_e1, b_e1[None, :], W_e2, b_e2[None, :])


# speedup vs baseline: 7.0857x; 1.0988x over previous
import functools

import jax
import jax.numpy as jnp
from jax import lax
from jax.experimental import pallas as pl
from jax.experimental.pallas import tpu as pltpu
from jax.experimental.pallas import tpu_sc as plsc

N_NODES = 10000
N_EDGES = 160000
M = 2 * N_EDGES
A = 128
E_FEAT = 112
ANG = 16
ZDIM = 384

NC = 2
NS = 16
NW = NC * NS
CE = 64
NCHK = N_EDGES // CE
NT = (NCHK + NW - 1) // NW
NPAIR = (NT + 1) // 2


def _sc_assemble(atom_fea, edge_aug, idx_pack):
    mesh = plsc.VectorSubcoreMesh(core_axis_name="c", subcore_axis_name="s")

    @functools.partial(
        pl.kernel,
        out_type=jax.ShapeDtypeStruct((N_EDGES, 6 * A), jnp.float32),
        mesh=mesh,
        scratch_types=[
            pltpu.VMEM((6 * CE,), jnp.int32),
            pltpu.VMEM((6 * CE,), jnp.int32),
            pltpu.VMEM((CE, A), jnp.float32),
            pltpu.VMEM((CE, A), jnp.float32),
            pltpu.VMEM((CE, A), jnp.float32),
            pltpu.VMEM((CE, A), jnp.float32),
            pltpu.VMEM((CE, A), jnp.float32),
            pltpu.VMEM((CE, A), jnp.float32),
            pltpu.VMEM((CE, A), jnp.float32),
            pltpu.VMEM((CE, A), jnp.float32),
            pltpu.VMEM((CE, A), jnp.float32),
            pltpu.VMEM((CE, A), jnp.float32),
            pltpu.VMEM((CE, A), jnp.float32),
            pltpu.VMEM((CE, A), jnp.float32),
            pltpu.SemaphoreType.DMA,
            pltpu.SemaphoreType.DMA,
            pltpu.SemaphoreType.DMA,
            pltpu.SemaphoreType.DMA,
        ],
    )
    def k(atom_hbm, edge_hbm, idx_hbm,
          z_hbm,
          xa0, xa1,
          b00, b01, b02, b03, b04, b05,
          b10, b11, b12, b13, b14, b15,
          sg0, sg1, sw0, sw1):
        wid = lax.axis_index("s") * NC + lax.axis_index("c")
        xall = (xa0, xa1)
        bufs = ((b00, b01, b02, b03, b04, b05),
                (b10, b11, b12, b13, b14, b15))
        sg = (sg0, sg1)
        sw = (sw0, sw1)
        tabs = (atom_hbm, atom_hbm, edge_hbm, atom_hbm, atom_hbm, edge_hbm)

        def chunk_of(t):
            return wid + t * NW

        def cond(t):
            return chunk_of(t) < NCHK

        def gathers_start(t, s):
            c = chunk_of(t)
            pltpu.sync_copy(idx_hbm.at[pl.ds(c * 6 * CE, 6 * CE)], xall[s])
            for kk in range(6):
                pltpu.async_copy(
                    tabs[kk].at[xall[s].at[pl.ds(kk * CE, CE)]],
                    bufs[s][kk], sg[s])

        def gathers_wait(s):
            for kk in range(6):
                pltpu.make_async_copy(
                    tabs[kk].at[xall[s].at[pl.ds(kk * CE, CE)]],
                    bufs[s][kk], sg[s]).wait()

        def writes_start(t, s):
            rows = pl.ds(chunk_of(t) * CE, CE)
            for kk in range(6):
                pltpu.async_copy(bufs[s][kk],
                                 z_hbm.at[rows, pl.ds(kk * A, A)], sw[s])

        def writes_wait(s):
            rows = pl.ds(0, CE)
            for kk in range(6):
                pltpu.make_async_copy(bufs[s][kk],
                                      z_hbm.at[rows, pl.ds(kk * A, A)],
                                      sw[s]).wait()

        gathers_start(0, 0)

        def pair(tt, carry):
            t0 = 2 * tt
            t1 = t0 + 1
            t2 = t0 + 2
            @pl.when(jnp.logical_and(cond(t1), t1 >= 3))
            def _():
                writes_wait(1)

            @pl.when(cond(t1))
            def _():
                gathers_start(t1, 1)

            @pl.when(cond(t0))
            def _():
                gathers_wait(0)
                writes_start(t0, 0)

            @pl.when(cond(t2))
            def _():
                writes_wait(0)
                gathers_start(t2, 0)

            @pl.when(cond(t1))
            def _():
                gathers_wait(1)
                writes_start(t1, 1)

            return carry

        lax.fori_loop(0, NPAIR, pair, 0)
        writes_wait(0)
        writes_wait(1)

    return k(atom_fea, edge_aug, idx_pack)


BE = 1280
DCOL = 2 * A + E_FEAT


def _tc_body(zp_ref, ang_ref, ef_ref, wfs_ref, wang_ref, bfs_ref,
             we1_ref, be1_ref, we2_ref, be2_ref, out_ref):
    zp = zp_ref[...]
    x = jnp.concatenate([zp[:, :ZDIM], zp[:, ZDIM:]], axis=0)
    ap = ang_ref[...]
    xang = jnp.concatenate([ap[:, :ANG], ap[:, ANG:]], axis=0)
    zz = jnp.dot(x, wfs_ref[...], preferred_element_type=jnp.float32)
    zz = zz + jnp.dot(xang, wang_ref[...], preferred_element_type=jnp.float32)
    zz = zz + bfs_ref[...]
    d = x[:, DCOL:DCOL + 1]
    expd = jnp.exp(d * d * (-1.0 / 18.0))
    g = jax.nn.sigmoid(zz[:, :A]) * jax.nn.softplus(zz[:, A:]) * expd
    cat = jnp.concatenate([g[:BE], g[BE:], ef_ref[...]], axis=-1)
    h = jnp.dot(cat, we1_ref[...], preferred_element_type=jnp.float32)
    h = jax.nn.silu(h + be1_ref[...])
    o = jnp.dot(h, we2_ref[...], preferred_element_type=jnp.float32)
    out_ref[...] = (o + be2_ref[...]).T


def _tc_compute(zp, ang, edge_fea, w_fs, w_ang, b_fs, w_e1, b_e1, w_e2, b_e2):
    nblk = N_EDGES // BE
    full = lambda shape: pl.BlockSpec(shape, lambda i: (0, 0))
    return pl.pallas_call(
        _tc_body,
        grid=(nblk,),
        in_specs=[
            pl.BlockSpec((BE, 6 * A), lambda i: (i, 0)),
            pl.BlockSpec((BE, 2 * ANG), lambda i: (i, 0)),
            pl.BlockSpec((BE, E_FEAT), lambda i: (i, 0)),
            full(w_fs.shape),
            full(w_ang.shape),
            full(b_fs.shape),
            full(w_e1.shape),
            full(b_e1.shape),
            full(w_e2.shape),
            full(b_e2.shape),
        ],
        out_specs=pl.BlockSpec((64, BE), lambda i: (0, i)),
        out_shape=jax.ShapeDtypeStruct((64, N_EDGES), jnp.float32),
        compiler_params=pltpu.CompilerParams(
            dimension_semantics=("arbitrary",),
        ),
    )(zp, ang, edge_fea, w_fs, w_ang, b_fs, w_e1, b_e1, w_e2, b_e2)


def kernel(atom_fea, edge_fea, sub_atom_idx, sub_edge_idx, sub_edge_ang,
           sub_index, distance, huge_structure, output_final_layer_neuron,
           W_f, b_f, W_s, b_s, W_e1, b_e1, W_e2, b_e2):
    sai = sub_atom_idx.astype(jnp.int32)
    ij = sub_edge_idx.astype(jnp.int32)
    i0e = sai[0::2, 0]
    i1e = sai[0::2, 1]
    ije = ij[0::2]
    i0o = sai[1::2, 0]
    i1o = sai[1::2, 1]
    ijo = ij[1::2]
    idx_pack = (jnp.stack([i0e, i1e, ije, i0o, i1o, ijo], axis=0)
                .reshape(6, NCHK, CE).transpose(1, 0, 2).reshape(-1))
    edge_aug = jnp.concatenate(
        [edge_fea, distance[:, None],
         jnp.zeros((N_EDGES, A - E_FEAT - 1), jnp.float32)], axis=1)
    zp = _sc_assemble(atom_fea, edge_aug, idx_pack)
    w_fs = jnp.concatenate([W_f, W_s], axis=1)
    w_fs_pad = w_fs.at[DCOL:, :].set(0.0)
    w_ang = w_fs[ZDIM - ANG:, :]
    b_fs = jnp.concatenate([b_f, b_s])[None, :]
    ap = sub_edge_ang.reshape(N_EDGES, 2 * ANG)
    out_t = _tc_compute(zp, ap, edge_fea, w_fs_pad, w_ang, b_fs,
                        W_e1, b_e1[None, :], W_e2, b_e2[None, :])
    return out_t.T


# speedup vs baseline: 8.6760x; 1.2244x over previous
import functools

import jax
import jax.numpy as jnp
from jax import lax
from jax.experimental import pallas as pl
from jax.experimental.pallas import tpu as pltpu
from jax.experimental.pallas import tpu_sc as plsc

N_NODES = 10000
N_EDGES = 160000
M = 2 * N_EDGES
A = 128
E_FEAT = 112
ANG = 16
ZDIM = 384

NC = 2
NS = 16
NW = NC * NS
CE = 64
NCHK = N_EDGES // CE
NT = (NCHK + NW - 1) // NW
NPAIR = (NT + 1) // 2


def _sc_assemble(atom_fea, edge_aug, i0, i1, ij):
    mesh = plsc.VectorSubcoreMesh(core_axis_name="c", subcore_axis_name="s")

    @functools.partial(
        pl.kernel,
        out_type=jax.ShapeDtypeStruct((N_EDGES, 6 * A), jnp.float32),
        mesh=mesh,
        scratch_types=[
            pltpu.VMEM((2 * CE,), jnp.int32),
            pltpu.VMEM((2 * CE,), jnp.int32),
            pltpu.VMEM((2 * CE,), jnp.int32),
            pltpu.VMEM((2 * CE,), jnp.int32),
            pltpu.VMEM((2 * CE,), jnp.int32),
            pltpu.VMEM((2 * CE,), jnp.int32),
            pltpu.VMEM((2 * CE, A), jnp.float32),
            pltpu.VMEM((2 * CE, A), jnp.float32),
            pltpu.VMEM((2 * CE, A), jnp.float32),
            pltpu.VMEM((2 * CE, A), jnp.float32),
            pltpu.VMEM((2 * CE, A), jnp.float32),
            pltpu.VMEM((2 * CE, A), jnp.float32),
            pltpu.SemaphoreType.DMA,
            pltpu.SemaphoreType.DMA,
            pltpu.SemaphoreType.DMA,
            pltpu.SemaphoreType.DMA,
            pltpu.SemaphoreType.DMA,
            pltpu.SemaphoreType.DMA,
        ],
    )
    def k(atom_hbm, edge_hbm, i0_hbm, i1_hbm, ij_hbm,
          z_hbm,
          x00, x01, x02, x10, x11, x12,
          b00, b01, b02, b10, b11, b12,
          sg0, sg1, sw0, sw1, si0, si1):
        wid = lax.axis_index("s") * NC + lax.axis_index("c")
        xraw = ((x00, x01, x02), (x10, x11, x12))
        bufs = ((b00, b01, b02), (b10, b11, b12))
        sg = (sg0, sg1)
        sw = (sw0, sw1)
        si = (si0, si1)
        idx_hbms = (i0_hbm, i1_hbm, ij_hbm)
        tabs = (atom_hbm, atom_hbm, edge_hbm)

        def chunk_of(t):
            return wid + t * NW

        def cond(t):
            return chunk_of(t) < NCHK

        def gathers_start(t, s):
            base = 2 * chunk_of(t) * CE
            cps = [pltpu.async_copy(idx_hbms[g].at[pl.ds(base, 2 * CE)],
                                    xraw[s][g], si[s]) for g in range(3)]
            for cp in cps:
                cp.wait()
            for g in range(3):
                pltpu.async_copy(tabs[g].at[xraw[s][g]], bufs[s][g], sg[s])

        def gathers_wait(s):
            for g in range(3):
                pltpu.make_async_copy(tabs[g].at[xraw[s][g]],
                                      bufs[s][g], sg[s]).wait()

        def writes_start(t, s):
            rows = pl.ds(chunk_of(t) * CE, CE)
            for g in range(3):
                pltpu.async_copy(bufs[s][g].reshape(CE, 2 * A),
                                 z_hbm.at[rows, pl.ds(g * 2 * A, 2 * A)],
                                 sw[s])

        def writes_wait(s):
            rows = pl.ds(0, CE)
            for g in range(3):
                pltpu.make_async_copy(bufs[s][g].reshape(CE, 2 * A),
                                      z_hbm.at[rows, pl.ds(g * 2 * A, 2 * A)],
                                      sw[s]).wait()

        gathers_start(0, 0)

        def pair(tt, carry):
            t0 = 2 * tt
            t1 = t0 + 1
            t2 = t0 + 2

            @pl.when(jnp.logical_and(cond(t1), t1 >= 3))
            def _():
                writes_wait(1)

            @pl.when(cond(t1))
            def _():
                gathers_start(t1, 1)

            @pl.when(cond(t0))
            def _():
                gathers_wait(0)
                writes_start(t0, 0)

            @pl.when(cond(t2))
            def _():
                writes_wait(0)
                gathers_start(t2, 0)

            @pl.when(cond(t1))
            def _():
                gathers_wait(1)
                writes_start(t1, 1)

            return carry

        lax.fori_loop(0, NPAIR, pair, 0)
        writes_wait(0)
        writes_wait(1)

    return k(atom_fea, edge_aug, i0, i1, ij)


BE = 1280
DCOL = 2 * A + E_FEAT


def _tc_body(zp_ref, ang_ref, ef_ref, wfs_ref, wang_ref, bfs_ref,
             we1_ref, be1_ref, we2_ref, be2_ref, out_ref):
    zp = zp_ref[...]
    xa = jnp.concatenate([zp[:, 0:A], zp[:, 2 * A:3 * A],
                          zp[:, 4 * A:5 * A]], axis=-1)
    xb = jnp.concatenate([zp[:, A:2 * A], zp[:, 3 * A:4 * A],
                          zp[:, 5 * A:6 * A]], axis=-1)
    x = jnp.concatenate([xa, xb], axis=0)
    ap = ang_ref[...]
    xang = jnp.concatenate([ap[:, :ANG], ap[:, ANG:]], axis=0)
    zz = jnp.dot(x, wfs_ref[...], preferred_element_type=jnp.float32)
    zz = zz + jnp.dot(xang, wang_ref[...], preferred_element_type=jnp.float32)
    zz = zz + bfs_ref[...]
    d = x[:, DCOL:DCOL + 1]
    expd = jnp.exp(d * d * (-1.0 / 18.0))
    g = jax.nn.sigmoid(zz[:, :A]) * jax.nn.softplus(zz[:, A:]) * expd
    cat = jnp.concatenate([g[:BE], g[BE:], ef_ref[...]], axis=-1)
    h = jnp.dot(cat, we1_ref[...], preferred_element_type=jnp.float32)
    h = jax.nn.silu(h + be1_ref[...])
    o = jnp.dot(h, we2_ref[...], preferred_element_type=jnp.float32)
    out_ref[...] = (o + be2_ref[...]).T


def _tc_compute(zp, ap, edge_fea, w_fs, w_ang, b_fs, w_e1, b_e1, w_e2, b_e2):
    nblk = N_EDGES // BE
    full = lambda shape: pl.BlockSpec(shape, lambda i: (0, 0))
    return pl.pallas_call(
        _tc_body,
        grid=(nblk,),
        in_specs=[
            pl.BlockSpec((BE, 2 * ZDIM), lambda i: (i, 0)),
            pl.BlockSpec((BE, 2 * ANG), lambda i: (i, 0)),
            pl.BlockSpec((BE, E_FEAT), lambda i: (i, 0)),
            full(w_fs.shape),
            full(w_ang.shape),
            full(b_fs.shape),
            full(w_e1.shape),
            full(b_e1.shape),
            full(w_e2.shape),
            full(b_e2.shape),
        ],
        out_specs=pl.BlockSpec((64, BE), lambda i: (0, i)),
        out_shape=jax.ShapeDtypeStruct((64, N_EDGES), jnp.float32),
        compiler_params=pltpu.CompilerParams(
            dimension_semantics=("arbitrary",),
        ),
    )(zp, ap, edge_fea, w_fs, w_ang, b_fs, w_e1, b_e1, w_e2, b_e2)


def kernel(atom_fea, edge_fea, sub_atom_idx, sub_edge_idx, sub_edge_ang,
           sub_index, distance, huge_structure, output_final_layer_neuron,
           W_f, b_f, W_s, b_s, W_e1, b_e1, W_e2, b_e2):
    sai = sub_atom_idx.astype(jnp.int32)
    ij = sub_edge_idx.astype(jnp.int32)
    i0 = sai[:, 0]
    i1 = sai[:, 1]
    edge_aug = jnp.concatenate(
        [edge_fea, distance[:, None],
         jnp.zeros((N_EDGES, A - E_FEAT - 1), jnp.float32)], axis=1)
    zp = _sc_assemble(atom_fea, edge_aug, i0, i1, ij)
    w_fs = jnp.concatenate([W_f, W_s], axis=1)
    w_fs_pad = w_fs.at[DCOL:, :].set(0.0)
    w_ang = w_fs[ZDIM - ANG:, :]
    b_fs = jnp.concatenate([b_f, b_s])[None, :]
    ap = sub_edge_ang.reshape(N_EDGES, 2 * ANG)
    out_t = _tc_compute(zp, ap, edge_fea, w_fs_pad, w_ang, b_fs,
                        W_e1, b_e1[None, :], W_e2, b_e2[None, :])
    return out_t.T


# speedup vs baseline: 9.0092x; 1.0384x over previous
import functools

import jax
import jax.numpy as jnp
from jax import lax
from jax.experimental import pallas as pl
from jax.experimental.pallas import tpu as pltpu
from jax.experimental.pallas import tpu_sc as plsc

N_NODES = 10000
N_EDGES = 160000
M = 2 * N_EDGES
A = 128
E_FEAT = 112
ANG = 16
ZDIM = 384

NC = 2
NS = 16
NW = NC * NS
CE = 64
NCHK = N_EDGES // CE
NT = (NCHK + NW - 1) // NW
NPAIR = (NT + 1) // 2


def _sc_assemble(atom_fea, edge_aug, i0, i1, ij):
    mesh = plsc.VectorSubcoreMesh(core_axis_name="c", subcore_axis_name="s")

    @functools.partial(
        pl.kernel,
        out_type=jax.ShapeDtypeStruct((N_EDGES, 6 * A), jnp.float32),
        mesh=mesh,
        scratch_types=[
            pltpu.VMEM((2 * CE,), jnp.int32),
            pltpu.VMEM((2 * CE,), jnp.int32),
            pltpu.VMEM((2 * CE,), jnp.int32),
            pltpu.VMEM((2 * CE,), jnp.int32),
            pltpu.VMEM((2 * CE,), jnp.int32),
            pltpu.VMEM((2 * CE,), jnp.int32),
            pltpu.VMEM((2 * CE, A), jnp.float32),
            pltpu.VMEM((2 * CE, A), jnp.float32),
            pltpu.VMEM((2 * CE, A), jnp.float32),
            pltpu.VMEM((2 * CE, A), jnp.float32),
            pltpu.VMEM((2 * CE, A), jnp.float32),
            pltpu.VMEM((2 * CE, A), jnp.float32),
            pltpu.SemaphoreType.DMA,
            pltpu.SemaphoreType.DMA,
            pltpu.SemaphoreType.DMA,
            pltpu.SemaphoreType.DMA,
            pltpu.SemaphoreType.DMA,
            pltpu.SemaphoreType.DMA,
        ],
    )
    def k(atom_hbm, edge_hbm, i0_hbm, i1_hbm, ij_hbm,
          z_hbm,
          x00, x01, x02, x10, x11, x12,
          b00, b01, b02, b10, b11, b12,
          sg0, sg1, sw0, sw1, si0, si1):
        wid = lax.axis_index("s") * NC + lax.axis_index("c")
        xraw = ((x00, x01, x02), (x10, x11, x12))
        bufs = ((b00, b01, b02), (b10, b11, b12))
        sg = (sg0, sg1)
        sw = (sw0, sw1)
        si = (si0, si1)
        idx_hbms = (i0_hbm, i1_hbm, ij_hbm)
        tabs = (atom_hbm, atom_hbm, edge_hbm)

        def chunk_of(t):
            return wid + t * NW

        def cond(t):
            return chunk_of(t) < NCHK

        def gathers_start(t, s):
            base = 2 * chunk_of(t) * CE
            cps = [pltpu.async_copy(idx_hbms[g].at[pl.ds(base, 2 * CE)],
                                    xraw[s][g], si[s]) for g in range(3)]
            for cp in cps:
                cp.wait()
            for g in range(3):
                pltpu.async_copy(tabs[g].at[xraw[s][g]], bufs[s][g], sg[s])

        def gathers_wait(s):
            for g in range(3):
                pltpu.make_async_copy(tabs[g].at[xraw[s][g]],
                                      bufs[s][g], sg[s]).wait()

        def writes_start(t, s):
            rows = pl.ds(chunk_of(t) * CE, CE)
            for g in range(3):
                pltpu.async_copy(bufs[s][g].reshape(CE, 2 * A),
                                 z_hbm.at[rows, pl.ds(g * 2 * A, 2 * A)],
                                 sw[s])

        def writes_wait(s):
            rows = pl.ds(0, CE)
            for g in range(3):
                pltpu.make_async_copy(bufs[s][g].reshape(CE, 2 * A),
                                      z_hbm.at[rows, pl.ds(g * 2 * A, 2 * A)],
                                      sw[s]).wait()

        gathers_start(0, 0)

        def pair(tt, carry):
            t0 = 2 * tt
            t1 = t0 + 1
            t2 = t0 + 2

            @pl.when(jnp.logical_and(cond(t1), t1 >= 3))
            def _():
                writes_wait(1)

            @pl.when(cond(t1))
            def _():
                gathers_start(t1, 1)

            @pl.when(cond(t0))
            def _():
                gathers_wait(0)
                writes_start(t0, 0)

            @pl.when(cond(t2))
            def _():
                writes_wait(0)
                gathers_start(t2, 0)

            @pl.when(cond(t1))
            def _():
                gathers_wait(1)
                writes_start(t1, 1)

            return carry

        lax.fori_loop(0, NPAIR, pair, 0)
        writes_wait(0)
        writes_wait(1)

    return k(atom_fea, edge_aug, i0, i1, ij)


BE = 3200
DCOL = 2 * A + E_FEAT


def _tc_body(zp_ref, ang_ref, ef_ref, wfs_ref, wang_ref, bfs_ref,
             we1_ref, be1_ref, we2_ref, be2_ref, out_ref):
    zp = zp_ref[...]
    xa = jnp.concatenate([zp[:, 0:A], zp[:, 2 * A:3 * A],
                          zp[:, 4 * A:5 * A]], axis=-1)
    xb = jnp.concatenate([zp[:, A:2 * A], zp[:, 3 * A:4 * A],
                          zp[:, 5 * A:6 * A]], axis=-1)
    x = jnp.concatenate([xa, xb], axis=0)
    ap = ang_ref[...]
    xang = jnp.concatenate([ap[:, :ANG], ap[:, ANG:]], axis=0)
    zz = jnp.dot(x, wfs_ref[...], preferred_element_type=jnp.float32)
    zz = zz + jnp.dot(xang, wang_ref[...], preferred_element_type=jnp.float32)
    zz = zz + bfs_ref[...]
    d = x[:, DCOL:DCOL + 1]
    expd = jnp.exp(d * d * (-1.0 / 18.0))
    g = jax.nn.sigmoid(zz[:, :A]) * jax.nn.softplus(zz[:, A:]) * expd
    cat = jnp.concatenate([g[:BE], g[BE:], ef_ref[...]], axis=-1)
    h = jnp.dot(cat, we1_ref[...], preferred_element_type=jnp.float32)
    h = jax.nn.silu(h + be1_ref[...])
    o = jnp.dot(h, we2_ref[...], preferred_element_type=jnp.float32)
    out_ref[...] = (o + be2_ref[...]).T


def _tc_compute(zp, ap, edge_fea, w_fs, w_ang, b_fs, w_e1, b_e1, w_e2, b_e2):
    nblk = N_EDGES // BE
    full = lambda shape: pl.BlockSpec(shape, lambda i: (0, 0))
    return pl.pallas_call(
        _tc_body,
        grid=(nblk,),
        in_specs=[
            pl.BlockSpec((BE, 2 * ZDIM), lambda i: (i, 0)),
            pl.BlockSpec((BE, 2 * ANG), lambda i: (i, 0)),
            pl.BlockSpec((BE, E_FEAT), lambda i: (i, 0)),
            full(w_fs.shape),
            full(w_ang.shape),
            full(b_fs.shape),
            full(w_e1.shape),
            full(b_e1.shape),
            full(w_e2.shape),
            full(b_e2.shape),
        ],
        out_specs=pl.BlockSpec((64, BE), lambda i: (0, i)),
        out_shape=jax.ShapeDtypeStruct((64, N_EDGES), jnp.float32),
        compiler_params=pltpu.CompilerParams(
            dimension_semantics=("parallel",),
        ),
    )(zp, ap, edge_fea, w_fs, w_ang, b_fs, w_e1, b_e1, w_e2, b_e2)


def kernel(atom_fea, edge_fea, sub_atom_idx, sub_edge_idx, sub_edge_ang,
           sub_index, distance, huge_structure, output_final_layer_neuron,
           W_f, b_f, W_s, b_s, W_e1, b_e1, W_e2, b_e2):
    sai = sub_atom_idx.astype(jnp.int32)
    ij = sub_edge_idx.astype(jnp.int32)
    i0 = sai[:, 0]
    i1 = sai[:, 1]
    edge_aug = jnp.concatenate(
        [edge_fea, distance[:, None],
         jnp.zeros((N_EDGES, A - E_FEAT - 1), jnp.float32)], axis=1)
    zp = _sc_assemble(atom_fea, edge_aug, i0, i1, ij)
    w_fs = jnp.concatenate([W_f, W_s], axis=1)
    w_fs_pad = w_fs.at[DCOL:, :].set(0.0)
    w_ang = w_fs[ZDIM - ANG:, :]
    b_fs = jnp.concatenate([b_f, b_s])[None, :]
    ap = sub_edge_ang.reshape(N_EDGES, 2 * ANG)
    out_t = _tc_compute(zp, ap, edge_fea, w_fs_pad, w_ang, b_fs,
                        W_e1, b_e1[None, :], W_e2, b_e2[None, :])
    return out_t.T
